# baseline (speedup 1.0000x reference)
import functools

import jax
import jax.numpy as jnp
from jax.experimental import pallas as pl
from jax.experimental.pallas import tpu as pltpu


def _im2col(xs, *, C, H, W):
    HW = H * W
    P = W + 1
    padded = jnp.pad(xs, ((0, 0), (P, P)))
    col = jax.lax.broadcasted_iota(jnp.int32, (C, HW), 1) % W

    taps = []
    for kh in (0, 1, 2):
        for kw in (0, 1, 2):
            s = (kh - 1) * W + (kw - 1)
            t = padded[:, P + s: P + s + HW]
            if kw == 0:
                t = jnp.where(col != 0, t, 0)
            elif kw == 2:
                t = jnp.where(col != W - 1, t, 0)
            taps.append(t)
    return jnp.concatenate(taps, axis=0)


def _conv3x3_kernel(x_ref, w_ref, b_ref, o_ref, *, C, H, W, NB, NC):
    HW = H * W
    Cout = o_ref.shape[-1]
    for g in range(0, NB, NC):
        patches = jnp.concatenate(
            [_im2col(x_ref[g + i].astype(jnp.bfloat16).reshape(C, HW),
                     C=C, H=H, W=W) for i in range(NC)], axis=1)
        acc = jax.lax.dot_general(
            patches, w_ref[...],
            dimension_numbers=(((0,), (1,)), ((), ())),
            preferred_element_type=jnp.float32)
        acc = acc + b_ref[...]
        res = jnp.maximum(acc, 0.0).astype(o_ref.dtype)
        for i in range(NC):
            o_ref[g + i] = res[i * HW:(i + 1) * HW].reshape(H, W, Cout)


def kernel(x, weight, bias):
    N, C, H, W = x.shape
    Cout = weight.shape[0]
    K = 9 * C

    wf = jnp.transpose(weight, (0, 2, 3, 1)).reshape(Cout, K).astype(jnp.bfloat16)
    b2 = bias.astype(jnp.float32).reshape(1, Cout)

    NB = 4 if N % 4 == 0 else (2 if N % 2 == 0 else 1)
    NC = min(NB, 2)
    out = pl.pallas_call(
        functools.partial(_conv3x3_kernel, C=C, H=H, W=W, NB=NB, NC=NC),
        out_shape=jax.ShapeDtypeStruct((N, H, W, Cout), x.dtype),
        grid=(N // NB,),
        in_specs=[
            pl.BlockSpec((NB, C, H, W), lambda n: (n, 0, 0, 0)),
            pl.BlockSpec((Cout, K), lambda n: (0, 0)),
            pl.BlockSpec((1, Cout), lambda n: (0, 0)),
        ],
        out_specs=pl.BlockSpec((NB, H, W, Cout), lambda n: (n, 0, 0, 0)),
        compiler_params=pltpu.CompilerParams(
            dimension_semantics=("parallel",),
            vmem_limit_bytes=64 * 1024 * 1024,
        ),
    )(x, wf, b2)
    return jnp.transpose(out, (0, 3, 1, 2))


# speedup vs baseline: 1.1705x; 1.1705x over previous
import functools

import jax
import jax.numpy as jnp
from jax.experimental import pallas as pl
from jax.experimental.pallas import tpu as pltpu


def _im2col(xs, *, C, H, W):
    HW = H * W
    P = W + 1
    padded = jnp.pad(xs, ((0, 0), (P, P)))
    col = jax.lax.broadcasted_iota(jnp.int32, (C, HW), 1) % W

    taps = []
    for kh in (0, 1, 2):
        for kw in (0, 1, 2):
            s = (kh - 1) * W + (kw - 1)
            t = padded[:, P + s: P + s + HW]
            if kw == 0:
                t = jnp.where(col != 0, t, 0)
            elif kw == 2:
                t = jnp.where(col != W - 1, t, 0)
            taps.append(t)
    return jnp.concatenate(taps, axis=0)


def _conv3x3_kernel(x_ref, w_ref, b_ref, o_ref, *, C, H, W, NB, NC):
    HW = H * W
    Cout = o_ref.shape[-1]
    for g in range(0, NB, NC):
        patches = jnp.concatenate(
            [_im2col(x_ref[g + i].astype(jnp.bfloat16).reshape(C, HW),
                     C=C, H=H, W=W) for i in range(NC)], axis=1)
        acc = jax.lax.dot_general(
            patches, w_ref[...],
            dimension_numbers=(((0,), (1,)), ((), ())),
            preferred_element_type=jnp.float32)
        acc = acc + b_ref[...]
        res = jnp.maximum(acc, 0.0).astype(o_ref.dtype)
        for i in range(NC):
            o_ref[g + i] = res[i * HW:(i + 1) * HW].reshape(H, W, Cout)


def kernel(x, weight, bias):
    N, C, H, W = x.shape
    Cout = weight.shape[0]
    K = 9 * C

    wf = jnp.transpose(weight, (0, 2, 3, 1)).reshape(Cout, K).astype(jnp.bfloat16)
    b2 = bias.astype(jnp.float32).reshape(1, Cout)

    NB = 2 if N % 2 == 0 else 1
    NC = min(NB, 2)
    out = pl.pallas_call(
        functools.partial(_conv3x3_kernel, C=C, H=H, W=W, NB=NB, NC=NC),
        out_shape=jax.ShapeDtypeStruct((N, H, W, Cout), x.dtype),
        grid=(N // NB,),
        in_specs=[
            pl.BlockSpec((NB, C, H, W), lambda n: (n, 0, 0, 0)),
            pl.BlockSpec((Cout, K), lambda n: (0, 0)),
            pl.BlockSpec((1, Cout), lambda n: (0, 0)),
        ],
        out_specs=pl.BlockSpec((NB, H, W, Cout), lambda n: (n, 0, 0, 0)),
        compiler_params=pltpu.CompilerParams(
            dimension_semantics=("parallel",),
            vmem_limit_bytes=64 * 1024 * 1024,
        ),
    )(x, wf, b2)
    return jnp.transpose(out, (0, 3, 1, 2))
